# Initial kernel scaffold; baseline (speedup 1.0000x reference)
import jax
import jax.numpy as jnp
from jax.experimental import pallas as pl


def kernel(x, edge_index, W_gcn, b_gcn, W_gat, att_src, att_dst, b_gat, W_sage_l, W_sage_r, b_sage, W1, b1, W2, b2):
    raise NotImplementedError("write your pallas kernel here")


# speedup vs baseline: 8.3177x; 8.3177x over previous
import functools

import jax
import jax.numpy as jnp
from jax import lax
from jax.experimental import pallas as pl
from jax.experimental.pallas import tpu as pltpu
from jax.experimental.pallas import tpu_sc as plsc

N = 10000
E = 160000
D = 128
HID = 128
HEADS = 4

NC = 2
NS = 16
NW = NC * NS
NT = N + 16
NPAD = 10240
EPT = E // NW
CHK = 200
CP = CHK + 8
NCHUNKS = EPT // CHK
STRIPE = NPAD // NS
ZROWS = 40


def _f32(*shape):
    return jax.ShapeDtypeStruct(shape, jnp.float32)


def _wid():
    return lax.axis_index("c") * NS + lax.axis_index("s")


def _iota16():
    return lax.iota(jnp.int32, 16)


def _zero_vmem(ref, nwords):
    z = jnp.zeros((16,), jnp.float32)

    def body(i, c):
        ref[pl.ds(i * 16, 16)] = z
        return c

    lax.fori_loop(0, nwords // 16, body, 0)


def _zero_zbuf(zbuf):
    z = jnp.zeros((16,), jnp.float32)

    def body(i, c):
        r = i // 8
        k = i % 8
        zbuf[r, pl.ds(k * 16, 16)] = z
        return c

    lax.fori_loop(0, ZROWS * 8, body, 0)


def _zero_stripe(acc, base, zbuf):
    def body(i, c):
        pltpu.sync_copy(zbuf, acc.at[pl.ds(base + i * ZROWS, ZROWS)])
        return c

    lax.fori_loop(0, STRIPE // ZROWS, body, 0)


def _load_chunk_hbm(hbm, off, buf, pad_val):
    pltpu.sync_copy(hbm.at[pl.ds(off, CHK)], buf.at[pl.ds(0, CHK)])
    t = buf[pl.ds(CHK - 8, 16)]
    pv = jnp.full((16,), pad_val, jnp.int32)
    buf[pl.ds(CHK - 8, 16)] = jnp.where(_iota16() < 8, t, pv)


def _sc_indeg_body(dst_hbm, out_hbm, dst_res, hist):
    w = _wid()
    _zero_vmem(hist, NT)
    pltpu.sync_copy(dst_hbm.at[pl.ds(w * EPT, EPT)], dst_res.at[pl.ds(0, EPT)])
    ones = jnp.ones((16,), jnp.float32)
    trash = jnp.full((16,), N, jnp.int32)
    lanes = _iota16()

    def body(g, c):
        idx = dst_res[pl.ds(g * 16, 16)]
        m = (g * 16 + lanes) < EPT
        plsc.addupdate_scatter(hist, [jnp.where(m, idx, trash)], ones)
        return c

    lax.fori_loop(0, (EPT + 15) // 16, body, 0)
    pltpu.sync_copy(hist.at[pl.ds(0, N)], out_hbm.at[pl.ds(w * N, N)])


def _sc_gcn_body(src_hbm, dst_hbm, xws_hbm, out_hbm,
            sbuf, dbuf, rowbuf, zbuf, acc, sem):
    cid = lax.axis_index("c")
    sid = lax.axis_index("s")
    w = cid * NS + sid
    base = sid * STRIPE
    _zero_zbuf(zbuf)
    _zero_stripe(acc, base, zbuf)
    plsc.subcore_barrier()

    def chunk(c, carry):
        eb = w * EPT + c * CHK
        _load_chunk_hbm(src_hbm, eb, sbuf, 0)
        _load_chunk_hbm(dst_hbm, eb, dbuf, N)
        pltpu.async_copy(xws_hbm.at[sbuf], rowbuf, sem).wait()
        pltpu.sync_copy(rowbuf, acc.at[dbuf], add=True)
        return carry

    lax.fori_loop(0, NCHUNKS, chunk, 0)
    plsc.subcore_barrier()
    pltpu.sync_copy(acc.at[pl.ds(base, STRIPE)],
                    out_hbm.at[cid, pl.ds(base, STRIPE)])


def _sc_att_body(src_hbm, dst_hbm, m_hbm, asT_hbm, adT_hbm,
            ex_hbm, den_hbm,
            sbuf, dbuf, exbuf, as_buf, ad_buf, mbuf, den_loc):
    cid = lax.axis_index("c")
    sid = lax.axis_index("s")
    w = cid * NS + sid
    pltpu.sync_copy(m_hbm, mbuf)

    for h in range(HEADS):
        pltpu.sync_copy(asT_hbm.at[pl.ds(h * N, N)], as_buf)
        pltpu.sync_copy(adT_hbm.at[pl.ds(h * N, N)], ad_buf.at[pl.ds(0, N)])
        ad_buf[pl.ds(N, 16)] = jnp.zeros((16,), jnp.float32)
        _zero_vmem(den_loc, NT)
        mh = mbuf[pl.ds(h * 16, 16)]

        def chunk(c, carry):
            eb = w * EPT + c * CHK
            _load_chunk_hbm(src_hbm, eb, sbuf, 0)
            _load_chunk_hbm(dst_hbm, eb, dbuf, N)

            def exg(g, cc):
                si = sbuf[pl.ds(g * 16, 16)]
                di = dbuf[pl.ds(g * 16, 16)]
                e = (plsc.load_gather(as_buf, [si])
                     + plsc.load_gather(ad_buf, [di]))
                e = jnp.where(e >= 0, e, 0.2 * e)
                ex = jnp.exp(e - mh)
                exbuf[pl.ds(g * 16, 16)] = ex
                plsc.addupdate_scatter(den_loc, [di], ex)
                return cc

            lax.fori_loop(0, CP // 16, exg, 0)
            pltpu.sync_copy(exbuf.at[pl.ds(0, CHK)],
                            ex_hbm.at[pl.ds(h * E + eb, CHK)])
            return carry

        lax.fori_loop(0, NCHUNKS, chunk, 0)
        pltpu.sync_copy(den_loc.at[pl.ds(0, N)],
                        den_hbm.at[pl.ds((w * HEADS + h) * N, N)])


def _sc_gatmm_body(src_hbm, dst_hbm, ex_hbm,
            xh0_hbm, xh1_hbm, xh2_hbm, xh3_hbm, num_hbm,
            sbuf, dbuf, exbuf, rowbuf, zbuf, acc, sem):
    cid = lax.axis_index("c")
    sid = lax.axis_index("s")
    w = cid * NS + sid
    base = sid * STRIPE
    _zero_zbuf(zbuf)
    xh_hbms = [xh0_hbm, xh1_hbm, xh2_hbm, xh3_hbm]
    zf = jnp.zeros((16,), jnp.float32)

    for h in range(HEADS):
        _zero_stripe(acc, base, zbuf)
        plsc.subcore_barrier()

        def chunk(c, carry):
            eb = w * EPT + c * CHK
            _load_chunk_hbm(src_hbm, eb, sbuf, 0)
            _load_chunk_hbm(dst_hbm, eb, dbuf, N)
            cp = pltpu.async_copy(xh_hbms[h].at[sbuf], rowbuf, sem)
            pltpu.sync_copy(ex_hbm.at[pl.ds(h * E + eb, CHK)],
                            exbuf.at[pl.ds(0, CHK)])
            t = exbuf[pl.ds(CHK - 8, 16)]
            exbuf[pl.ds(CHK - 8, 16)] = jnp.where(_iota16() < 8, t, zf)
            cp.wait()

            def wrow(i, cc):
                ww = plsc.load_gather(exbuf, [jnp.zeros((16,), jnp.int32) + i])
                for r in range(D // 16):
                    rowbuf[i, pl.ds(r * 16, 16)] = (
                        rowbuf[i, pl.ds(r * 16, 16)] * ww)
                return cc

            lax.fori_loop(0, CP, wrow, 0)
            pltpu.sync_copy(rowbuf, acc.at[dbuf], add=True)
            return carry

        lax.fori_loop(0, NCHUNKS, chunk, 0)
        plsc.subcore_barrier()
        pltpu.sync_copy(acc.at[pl.ds(base, STRIPE)],
                        num_hbm.at[cid, h, pl.ds(base, STRIPE)])


def _sc_sage_body(src_hbm, dst_hbm, h20_hbm, h21_hbm, h22_hbm, h23_hbm, out_hbm,
             sbuf, dbuf, rowbuf, zbuf, acc, sem):
    cid = lax.axis_index("c")
    sid = lax.axis_index("s")
    w = cid * NS + sid
    base = sid * STRIPE
    _zero_zbuf(zbuf)
    h2_hbms = [h20_hbm, h21_hbm, h22_hbm, h23_hbm]

    for h in range(HEADS):
        _zero_stripe(acc, base, zbuf)
        plsc.subcore_barrier()

        def chunk(c, carry):
            eb = w * EPT + c * CHK
            _load_chunk_hbm(src_hbm, eb, sbuf, 0)
            _load_chunk_hbm(dst_hbm, eb, dbuf, N)
            pltpu.async_copy(h2_hbms[h].at[sbuf], rowbuf, sem).wait()
            pltpu.sync_copy(rowbuf, acc.at[dbuf], add=True)
            return carry

        lax.fori_loop(0, NCHUNKS, chunk, 0)
        plsc.subcore_barrier()
        pltpu.sync_copy(acc.at[pl.ds(base, STRIPE)],
                        out_hbm.at[cid, h, pl.ds(base, STRIPE)])


def _sc_mlp_body(src_hbm, dst_hbm, u_hbm, v_hbm, w2_hbm, b2_hbm, out_hbm,
            sbuf, dbuf, ubuf, vbuf, accbuf, predbuf,
            w2buf, b2buf, sem1, sem2):
    cid = lax.axis_index("c")
    sid = lax.axis_index("s")
    w = cid * NS + sid
    pltpu.sync_copy(w2_hbm, w2buf)
    pltpu.sync_copy(b2_hbm, b2buf)
    w2v = [w2buf[pl.ds(r * 16, 16)] for r in range(D // 16)]
    b2v = b2buf[...]
    lanes = _iota16()

    def chunk(c, carry):
        eb = w * EPT + c * CHK
        _load_chunk_hbm(src_hbm, eb, sbuf, 0)
        _load_chunk_hbm(dst_hbm, eb, dbuf, 0)
        cp1 = pltpu.async_copy(u_hbm.at[sbuf], ubuf, sem1)
        cp2 = pltpu.async_copy(v_hbm.at[dbuf], vbuf, sem2)
        cp1.wait()
        cp2.wait()

        def edge(i, cc):
            acc = jnp.zeros((16,), jnp.float32)
            for r in range(D // 16):
                z = jnp.maximum(
                    ubuf[i, pl.ds(r * 16, 16)] + vbuf[i, pl.ds(r * 16, 16)], 0.0)
                acc = acc + z * w2v[r]
            accbuf[pl.ds(i * 16, 16)] = acc
            return cc

        lax.fori_loop(0, CP, edge, 0)

        def grp(g, cc):
            tot = jnp.zeros((16,), jnp.float32)
            rowbase = (g * 16 + lanes) * 16
            for r in range(16):
                tot = tot + plsc.load_gather(accbuf, [rowbase + r])
            s = tot + b2v
            predbuf[pl.ds(g * 16, 16)] = 1.0 / (1.0 + jnp.exp(-s))
            return cc

        lax.fori_loop(0, CP // 16, grp, 0)
        pltpu.sync_copy(predbuf.at[pl.ds(0, CHK)],
                        out_hbm.at[pl.ds(w * EPT + c * CHK, CHK)])
        return carry

    lax.fori_loop(0, NCHUNKS, chunk, 0)


@functools.lru_cache(maxsize=None)
def _sc_kernels():
    mesh = plsc.VectorSubcoreMesh(core_axis_name="c", subcore_axis_name="s")
    cp = pltpu.CompilerParams(needs_layout_passes=False)
    sc_indeg = pl.kernel(
        _sc_indeg_body,
        out_type=_f32(NW * N),
        mesh=mesh,
        compiler_params=cp,
        scratch_types=[
            pltpu.VMEM((EPT + 16,), jnp.int32),
            pltpu.VMEM((NT,), jnp.float32),
        ],
    )
    sc_gcn = pl.kernel(
        _sc_gcn_body,
        out_type=_f32(NC, NPAD, D),
        mesh=mesh,
        compiler_params=cp,
        scratch_types=[
            pltpu.VMEM((CP,), jnp.int32),
            pltpu.VMEM((CP,), jnp.int32),
            pltpu.VMEM((CP, D), jnp.float32),
            pltpu.VMEM((ZROWS, D), jnp.float32),
            pltpu.VMEM_SHARED((NPAD, D), jnp.float32),
            pltpu.SemaphoreType.DMA,
        ],
    )
    sc_att = pl.kernel(
        _sc_att_body,
        out_type=(_f32(HEADS * E), _f32(NW * HEADS * N)),
        mesh=mesh,
        compiler_params=cp,
        scratch_types=[
            pltpu.VMEM((CP,), jnp.int32),
            pltpu.VMEM((CP,), jnp.int32),
            pltpu.VMEM((CP,), jnp.float32),
            pltpu.VMEM((N,), jnp.float32),
            pltpu.VMEM((NT,), jnp.float32),
            pltpu.VMEM((HEADS * 16,), jnp.float32),
            pltpu.VMEM((NT,), jnp.float32),
        ],
    )
    sc_gatmm = pl.kernel(
        _sc_gatmm_body,
        out_type=_f32(NC, HEADS, NPAD, D),
        mesh=mesh,
        compiler_params=cp,
        scratch_types=[
            pltpu.VMEM((CP,), jnp.int32),
            pltpu.VMEM((CP,), jnp.int32),
            pltpu.VMEM((CP,), jnp.float32),
            pltpu.VMEM((CP, D), jnp.float32),
            pltpu.VMEM((ZROWS, D), jnp.float32),
            pltpu.VMEM_SHARED((NPAD, D), jnp.float32),
            pltpu.SemaphoreType.DMA,
        ],
    )
    sc_sage = pl.kernel(
        _sc_sage_body,
        out_type=_f32(NC, HEADS, NPAD, D),
        mesh=mesh,
        compiler_params=cp,
        scratch_types=[
            pltpu.VMEM((CP,), jnp.int32),
            pltpu.VMEM((CP,), jnp.int32),
            pltpu.VMEM((CP, D), jnp.float32),
            pltpu.VMEM((ZROWS, D), jnp.float32),
            pltpu.VMEM_SHARED((NPAD, D), jnp.float32),
            pltpu.SemaphoreType.DMA,
        ],
    )
    sc_mlp = pl.kernel(
        _sc_mlp_body,
        out_type=_f32(E),
        mesh=mesh,
        compiler_params=cp,
        scratch_types=[
            pltpu.VMEM((CP,), jnp.int32),
            pltpu.VMEM((CP,), jnp.int32),
            pltpu.VMEM((CP, D), jnp.float32),
            pltpu.VMEM((CP, D), jnp.float32),
            pltpu.VMEM((CP * 16,), jnp.float32),
            pltpu.VMEM((CP,), jnp.float32),
            pltpu.VMEM((D,), jnp.float32),
            pltpu.VMEM((16,), jnp.float32),
            pltpu.SemaphoreType.DMA,
            pltpu.SemaphoreType.DMA,
        ],
    )
    return sc_indeg, sc_gcn, sc_att, sc_gatmm, sc_sage, sc_mlp


BM = 512
GRID = (N + BM - 1) // BM


def _dot(a, b):
    return jnp.dot(a, b, preferred_element_type=jnp.float32)


def _tc1_body(x_ref, w_ref, degp_ref, xw_ref, xws_ref):
    indeg = jnp.sum(degp_ref[...], axis=0)
    dinv = lax.rsqrt(indeg + 1.0)
    xw = _dot(x_ref[...], w_ref[...])
    xw_ref[...] = xw
    xws_ref[...] = xw * dinv[:, None]


def _tc1(x, w_gcn, degp):
    return pl.pallas_call(
        _tc1_body,
        grid=(GRID,),
        in_specs=[
            pl.BlockSpec((BM, D), lambda i: (i, 0)),
            pl.BlockSpec((D, HID), lambda i: (0, 0)),
            pl.BlockSpec((NW, BM), lambda i: (0, i)),
        ],
        out_specs=[
            pl.BlockSpec((BM, HID), lambda i: (i, 0)),
            pl.BlockSpec((BM, HID), lambda i: (i, 0)),
        ],
        out_shape=[_f32(N, HID), _f32(N, HID)],
    )(x, w_gcn, degp)


def _tc3_body(p_ref, xw_ref, degp_ref, bgcn_ref, wgat_ref, asrc_ref, adst_ref,
              xh0_ref, xh1_ref, xh2_ref, xh3_ref, asT_ref, adT_ref):
    indeg = jnp.sum(degp_ref[...], axis=0)
    dinv = lax.rsqrt(indeg + 1.0)
    ps = p_ref[0] + p_ref[1]
    h1 = jax.nn.relu(dinv[:, None] * ps + (dinv * dinv)[:, None] * xw_ref[...]
                     + bgcn_ref[...])
    xh = _dot(h1, wgat_ref[...])
    asrc = asrc_ref[...]
    adst = adst_ref[...]
    xh_refs = [xh0_ref, xh1_ref, xh2_ref, xh3_ref]
    a_s = []
    a_d = []
    for h in range(HEADS):
        xhh = xh[:, h * HID:(h + 1) * HID]
        xh_refs[h][...] = xhh
        a_s.append(jnp.sum(xhh * asrc[h][None, :], axis=1).reshape(1, BM))
        a_d.append(jnp.sum(xhh * adst[h][None, :], axis=1).reshape(1, BM))
    asT_ref[...] = jnp.concatenate(a_s, axis=0)
    adT_ref[...] = jnp.concatenate(a_d, axis=0)


def _tc3(p, xw, degp, b_gcn, w_gat, att_src, att_dst):
    return pl.pallas_call(
        _tc3_body,
        grid=(GRID,),
        in_specs=[
            pl.BlockSpec((NC, BM, HID), lambda i: (0, i, 0)),
            pl.BlockSpec((BM, HID), lambda i: (i, 0)),
            pl.BlockSpec((NW, BM), lambda i: (0, i)),
            pl.BlockSpec((1, HID), lambda i: (0, 0)),
            pl.BlockSpec((HID, HEADS * HID), lambda i: (0, 0)),
            pl.BlockSpec((HEADS, HID), lambda i: (0, 0)),
            pl.BlockSpec((HEADS, HID), lambda i: (0, 0)),
        ],
        out_specs=[pl.BlockSpec((BM, HID), lambda i: (i, 0))] * 4
        + [pl.BlockSpec((HEADS, BM), lambda i: (0, i))] * 2,
        out_shape=[_f32(N, HID)] * 4 + [_f32(HEADS, N)] * 2,
    )(p, xw, degp, b_gcn, w_gat, att_src, att_dst)


def _tc5_body(num_ref, denp_ref, asT_ref, adT_ref, m_ref,
              xh0_ref, xh1_ref, xh2_ref, xh3_ref, bgat_ref,
              h20_ref, h21_ref, h22_ref, h23_ref):
    denp = jnp.sum(denp_ref[...], axis=0)
    a = asT_ref[...] + adT_ref[...]
    e = jnp.where(a >= 0, a, 0.2 * a) - m_ref[...]
    exs = jnp.exp(e)
    xh_refs = [xh0_ref, xh1_ref, xh2_ref, xh3_ref]
    h2_refs = [h20_ref, h21_ref, h22_ref, h23_ref]
    bgat = bgat_ref[...]
    for h in range(HEADS):
        den = denp[h] + exs[h]
        nm = num_ref[0, h] + num_ref[1, h] + exs[h][:, None] * xh_refs[h][...]
        h2 = nm / (den[:, None] + 1e-16) + bgat[:, h * HID:(h + 1) * HID]
        h2_refs[h][...] = jax.nn.relu(h2)


def _tc5(num, denp, asT, adT, m, xhs, b_gat):
    return pl.pallas_call(
        _tc5_body,
        grid=(GRID,),
        in_specs=[
            pl.BlockSpec((NC, HEADS, BM, HID), lambda i: (0, 0, i, 0)),
            pl.BlockSpec((NW, HEADS, BM), lambda i: (0, 0, i)),
            pl.BlockSpec((HEADS, BM), lambda i: (0, i)),
            pl.BlockSpec((HEADS, BM), lambda i: (0, i)),
            pl.BlockSpec((HEADS, 1), lambda i: (0, 0)),
        ] + [pl.BlockSpec((BM, HID), lambda i: (i, 0))] * 4
        + [pl.BlockSpec((1, HEADS * HID), lambda i: (0, 0))],
        out_specs=[pl.BlockSpec((BM, HID), lambda i: (i, 0))] * 4,
        out_shape=[_f32(N, HID)] * 4,
    )(num, denp, asT, adT, m, *xhs, b_gat)


def _tc7_body(sp_ref, h20_ref, h21_ref, h22_ref, h23_ref, degp_ref,
              wsl_ref, wsr_ref, bsage_ref, w1_ref, b1_ref, u_ref, v_ref):
    indeg = jnp.sum(degp_ref[...], axis=0)
    invd = 1.0 / jnp.maximum(indeg, 1.0)
    h2_refs = [h20_ref, h21_ref, h22_ref, h23_ref]
    acc = jnp.broadcast_to(bsage_ref[...], (BM, HID))
    wsl = wsl_ref[...]
    wsr = wsr_ref[...]
    for h in range(HEADS):
        aggh = (sp_ref[0, h] + sp_ref[1, h]) * invd[:, None]
        acc = acc + _dot(aggh, wsl[h * HID:(h + 1) * HID])
        acc = acc + _dot(h2_refs[h][...], wsr[h * HID:(h + 1) * HID])
    h3 = jax.nn.relu(acc)
    w1 = w1_ref[...]
    u_ref[...] = _dot(h3, w1[:HID])
    v_ref[...] = _dot(h3, w1[HID:]) + b1_ref[...]


def _tc7(sp, h2s, degp, w_sage_l, w_sage_r, b_sage, w1, b1):
    return pl.pallas_call(
        _tc7_body,
        grid=(GRID,),
        in_specs=[
            pl.BlockSpec((NC, HEADS, BM, HID), lambda i: (0, 0, i, 0)),
        ] + [pl.BlockSpec((BM, HID), lambda i: (i, 0))] * 4 + [
            pl.BlockSpec((NW, BM), lambda i: (0, i)),
            pl.BlockSpec((HEADS * HID, HID), lambda i: (0, 0)),
            pl.BlockSpec((HEADS * HID, HID), lambda i: (0, 0)),
            pl.BlockSpec((1, HID), lambda i: (0, 0)),
            pl.BlockSpec((2 * HID, HID), lambda i: (0, 0)),
            pl.BlockSpec((1, HID), lambda i: (0, 0)),
        ],
        out_specs=[pl.BlockSpec((BM, HID), lambda i: (i, 0))] * 2,
        out_shape=[_f32(N, HID)] * 2,
    )(sp, *h2s, degp, w_sage_l, w_sage_r, b_sage, w1, b1)


def kernel(x, edge_index, W_gcn, b_gcn, W_gat, att_src, att_dst, b_gat,
           W_sage_l, W_sage_r, b_sage, W1, b1, W2, b2):
    src = edge_index[0]
    dst = edge_index[1]
    _sc_indeg, _sc_gcn, _sc_att, _sc_gatmm, _sc_sage, _sc_mlp = _sc_kernels()

    degp = _sc_indeg(dst).reshape(NW, N)
    xw, xws = _tc1(x, W_gcn, degp)
    p = _sc_gcn(src, dst, xws)
    xh0, xh1, xh2, xh3, asT, adT = _tc3(
        p[:, :N], xw, degp, b_gcn.reshape(1, -1), W_gat, att_src, att_dst)
    M = jnp.maximum(jnp.max(asT, axis=1) + jnp.max(adT, axis=1), 0.0)
    m64 = jnp.repeat(M, 16)
    exf, denf = _sc_att(src, dst, m64, asT.reshape(-1), adT.reshape(-1))
    num = _sc_gatmm(src, dst, exf, xh0, xh1, xh2, xh3)
    denp = denf.reshape(NW, HEADS, N)
    h2s = _tc5(num[:, :, :N], denp, asT, adT, M.reshape(HEADS, 1),
               [xh0, xh1, xh2, xh3], b_gat.reshape(1, -1))
    sp = _sc_sage(src, dst, *h2s)
    u, v = _tc7(sp[:, :, :N], h2s, degp, W_sage_l, W_sage_r,
                b_sage.reshape(1, -1), W1, b1.reshape(1, -1))
    pred = _sc_mlp(src, dst, u, v, W2.reshape(-1), jnp.pad(b2, (0, 15)))
    return pred


# speedup vs baseline: 8.3804x; 1.0075x over previous
import functools

import jax
import jax.numpy as jnp
from jax import lax
from jax.experimental import pallas as pl
from jax.experimental.pallas import tpu as pltpu
from jax.experimental.pallas import tpu_sc as plsc

N = 10000
E = 160000
D = 128
HID = 128
HEADS = 4

NC = 2
NS = 16
NW = NC * NS
NT = N + 16
NPAD = 10240
EPT = E // NW
CHK = 200
CP = CHK + 8
NCHUNKS = EPT // CHK
STRIPE = NPAD // NS
ZROWS = 40


def _f32(*shape):
    return jax.ShapeDtypeStruct(shape, jnp.float32)


def _wid():
    return lax.axis_index("c") * NS + lax.axis_index("s")


def _iota16():
    return lax.iota(jnp.int32, 16)


def _zero_vmem(ref, nwords):
    z = jnp.zeros((16,), jnp.float32)

    def body(i, c):
        ref[pl.ds(i * 16, 16)] = z
        return c

    lax.fori_loop(0, nwords // 16, body, 0)


def _zero_zbuf(zbuf):
    z = jnp.zeros((16,), jnp.float32)

    def body(i, c):
        r = i // 8
        k = i % 8
        zbuf[r, pl.ds(k * 16, 16)] = z
        return c

    lax.fori_loop(0, ZROWS * 8, body, 0)


def _zero_stripe(acc, base, zbuf):
    def body(i, c):
        pltpu.sync_copy(zbuf, acc.at[pl.ds(base + i * ZROWS, ZROWS)])
        return c

    lax.fori_loop(0, STRIPE // ZROWS, body, 0)


def _load_chunk_hbm(hbm, off, buf, pad_val):
    pltpu.sync_copy(hbm.at[pl.ds(off, CHK)], buf.at[pl.ds(0, CHK)])
    t = buf[pl.ds(CHK - 8, 16)]
    pv = jnp.full((16,), pad_val, jnp.int32)
    buf[pl.ds(CHK - 8, 16)] = jnp.where(_iota16() < 8, t, pv)


def _sc_indeg_body(dst_hbm, out_hbm, dst_res, hist):
    w = _wid()
    _zero_vmem(hist, NT)
    pltpu.sync_copy(dst_hbm.at[pl.ds(w * EPT, EPT)], dst_res.at[pl.ds(0, EPT)])
    ones = jnp.ones((16,), jnp.float32)
    trash = jnp.full((16,), N, jnp.int32)
    lanes = _iota16()

    @plsc.parallel_loop(0, (EPT + 15) // 16, unroll=4)
    def body(g):
        idx = dst_res[pl.ds(g * 16, 16)]
        m = (g * 16 + lanes) < EPT
        plsc.addupdate_scatter(hist, [jnp.where(m, idx, trash)], ones)
    pltpu.sync_copy(hist.at[pl.ds(0, N)], out_hbm.at[pl.ds(w * N, N)])


def _sc_gcn_body(src_hbm, dst_hbm, xws_hbm, out_hbm,
            sbuf, dbuf, rowbuf, zbuf, acc, sem):
    cid = lax.axis_index("c")
    sid = lax.axis_index("s")
    w = cid * NS + sid
    base = sid * STRIPE
    _zero_zbuf(zbuf)
    _zero_stripe(acc, base, zbuf)
    plsc.subcore_barrier()

    def chunk(c, carry):
        eb = w * EPT + c * CHK
        _load_chunk_hbm(src_hbm, eb, sbuf, 0)
        _load_chunk_hbm(dst_hbm, eb, dbuf, N)
        pltpu.async_copy(xws_hbm.at[sbuf], rowbuf, sem).wait()
        pltpu.sync_copy(rowbuf, acc.at[dbuf], add=True)
        return carry

    lax.fori_loop(0, NCHUNKS, chunk, 0)
    plsc.subcore_barrier()
    pltpu.sync_copy(acc.at[pl.ds(base, STRIPE)],
                    out_hbm.at[cid, pl.ds(base, STRIPE)])


def _sc_att_body(src_hbm, dst_hbm, m_hbm, asT_hbm, adT_hbm,
            ex_hbm, den_hbm,
            sbuf, dbuf, exbuf, as_buf, ad_buf, mbuf, den_loc):
    cid = lax.axis_index("c")
    sid = lax.axis_index("s")
    w = cid * NS + sid
    pltpu.sync_copy(m_hbm, mbuf)

    for h in range(HEADS):
        pltpu.sync_copy(asT_hbm.at[pl.ds(h * N, N)], as_buf)
        pltpu.sync_copy(adT_hbm.at[pl.ds(h * N, N)], ad_buf.at[pl.ds(0, N)])
        ad_buf[pl.ds(N, 16)] = jnp.zeros((16,), jnp.float32)
        _zero_vmem(den_loc, NT)
        mh = mbuf[pl.ds(h * 16, 16)]

        def chunk(c, carry):
            eb = w * EPT + c * CHK
            _load_chunk_hbm(src_hbm, eb, sbuf, 0)
            _load_chunk_hbm(dst_hbm, eb, dbuf, N)

            @plsc.parallel_loop(0, CP // 16, unroll=4)
            def exg(g):
                si = sbuf[pl.ds(g * 16, 16)]
                di = dbuf[pl.ds(g * 16, 16)]
                e = (plsc.load_gather(as_buf, [si])
                     + plsc.load_gather(ad_buf, [di]))
                e = jnp.where(e >= 0, e, 0.2 * e)
                ex = jnp.exp(e - mh)
                exbuf[pl.ds(g * 16, 16)] = ex
                plsc.addupdate_scatter(den_loc, [di], ex)
            pltpu.sync_copy(exbuf.at[pl.ds(0, CHK)],
                            ex_hbm.at[pl.ds(h * E + eb, CHK)])
            return carry

        lax.fori_loop(0, NCHUNKS, chunk, 0)
        pltpu.sync_copy(den_loc.at[pl.ds(0, N)],
                        den_hbm.at[pl.ds((w * HEADS + h) * N, N)])


def _sc_gatmm_body(src_hbm, dst_hbm, ex_hbm,
            xh0_hbm, xh1_hbm, xh2_hbm, xh3_hbm, num_hbm,
            sbuf, dbuf, exbuf, rowbuf, zbuf, acc, sem):
    cid = lax.axis_index("c")
    sid = lax.axis_index("s")
    w = cid * NS + sid
    base = sid * STRIPE
    _zero_zbuf(zbuf)
    xh_hbms = [xh0_hbm, xh1_hbm, xh2_hbm, xh3_hbm]
    zf = jnp.zeros((16,), jnp.float32)

    for h in range(HEADS):
        _zero_stripe(acc, base, zbuf)
        plsc.subcore_barrier()

        def chunk(c, carry):
            eb = w * EPT + c * CHK
            _load_chunk_hbm(src_hbm, eb, sbuf, 0)
            _load_chunk_hbm(dst_hbm, eb, dbuf, N)
            cp = pltpu.async_copy(xh_hbms[h].at[sbuf], rowbuf, sem)
            pltpu.sync_copy(ex_hbm.at[pl.ds(h * E + eb, CHK)],
                            exbuf.at[pl.ds(0, CHK)])
            t = exbuf[pl.ds(CHK - 8, 16)]
            exbuf[pl.ds(CHK - 8, 16)] = jnp.where(_iota16() < 8, t, zf)
            cp.wait()

            @plsc.parallel_loop(0, CP, unroll=4)
            def wrow(i):
                ww = plsc.load_gather(exbuf, [jnp.zeros((16,), jnp.int32) + i])
                for r in range(D // 16):
                    rowbuf[i, pl.ds(r * 16, 16)] = (
                        rowbuf[i, pl.ds(r * 16, 16)] * ww)
            pltpu.sync_copy(rowbuf, acc.at[dbuf], add=True)
            return carry

        lax.fori_loop(0, NCHUNKS, chunk, 0)
        plsc.subcore_barrier()
        pltpu.sync_copy(acc.at[pl.ds(base, STRIPE)],
                        num_hbm.at[cid, h, pl.ds(base, STRIPE)])


def _sc_sage_body(src_hbm, dst_hbm, h20_hbm, h21_hbm, h22_hbm, h23_hbm, out_hbm,
             sbuf, dbuf, rowbuf, zbuf, acc, sem):
    cid = lax.axis_index("c")
    sid = lax.axis_index("s")
    w = cid * NS + sid
    base = sid * STRIPE
    _zero_zbuf(zbuf)
    h2_hbms = [h20_hbm, h21_hbm, h22_hbm, h23_hbm]

    for h in range(HEADS):
        _zero_stripe(acc, base, zbuf)
        plsc.subcore_barrier()

        def chunk(c, carry):
            eb = w * EPT + c * CHK
            _load_chunk_hbm(src_hbm, eb, sbuf, 0)
            _load_chunk_hbm(dst_hbm, eb, dbuf, N)
            pltpu.async_copy(h2_hbms[h].at[sbuf], rowbuf, sem).wait()
            pltpu.sync_copy(rowbuf, acc.at[dbuf], add=True)
            return carry

        lax.fori_loop(0, NCHUNKS, chunk, 0)
        plsc.subcore_barrier()
        pltpu.sync_copy(acc.at[pl.ds(base, STRIPE)],
                        out_hbm.at[cid, h, pl.ds(base, STRIPE)])


def _sc_mlp_body(src_hbm, dst_hbm, u_hbm, v_hbm, w2_hbm, b2_hbm, out_hbm,
            sbuf, dbuf, ubuf, vbuf, accbuf, predbuf,
            w2buf, b2buf, sem1, sem2):
    cid = lax.axis_index("c")
    sid = lax.axis_index("s")
    w = cid * NS + sid
    pltpu.sync_copy(w2_hbm, w2buf)
    pltpu.sync_copy(b2_hbm, b2buf)
    w2v = [w2buf[pl.ds(r * 16, 16)] for r in range(D // 16)]
    b2v = b2buf[...]
    lanes = _iota16()

    def chunk(c, carry):
        eb = w * EPT + c * CHK
        _load_chunk_hbm(src_hbm, eb, sbuf, 0)
        _load_chunk_hbm(dst_hbm, eb, dbuf, 0)
        cp1 = pltpu.async_copy(u_hbm.at[sbuf], ubuf, sem1)
        cp2 = pltpu.async_copy(v_hbm.at[dbuf], vbuf, sem2)
        cp1.wait()
        cp2.wait()

        @plsc.parallel_loop(0, CP, unroll=4)
        def edge(i):
            acc = jnp.zeros((16,), jnp.float32)
            for r in range(D // 16):
                z = jnp.maximum(
                    ubuf[i, pl.ds(r * 16, 16)] + vbuf[i, pl.ds(r * 16, 16)], 0.0)
                acc = acc + z * w2v[r]
            accbuf[pl.ds(i * 16, 16)] = acc

        @plsc.parallel_loop(0, CP // 16, unroll=2)
        def grp(g):
            tot = jnp.zeros((16,), jnp.float32)
            rowbase = (g * 16 + lanes) * 16
            for r in range(16):
                tot = tot + plsc.load_gather(accbuf, [rowbase + r])
            s = tot + b2v
            predbuf[pl.ds(g * 16, 16)] = 1.0 / (1.0 + jnp.exp(-s))
        pltpu.sync_copy(predbuf.at[pl.ds(0, CHK)],
                        out_hbm.at[pl.ds(w * EPT + c * CHK, CHK)])
        return carry

    lax.fori_loop(0, NCHUNKS, chunk, 0)


@functools.lru_cache(maxsize=None)
def _sc_kernels():
    mesh = plsc.VectorSubcoreMesh(core_axis_name="c", subcore_axis_name="s")
    cp = pltpu.CompilerParams(needs_layout_passes=False)
    sc_indeg = pl.kernel(
        _sc_indeg_body,
        out_type=_f32(NW * N),
        mesh=mesh,
        compiler_params=cp,
        scratch_types=[
            pltpu.VMEM((EPT + 16,), jnp.int32),
            pltpu.VMEM((NT,), jnp.float32),
        ],
    )
    sc_gcn = pl.kernel(
        _sc_gcn_body,
        out_type=_f32(NC, NPAD, D),
        mesh=mesh,
        compiler_params=cp,
        scratch_types=[
            pltpu.VMEM((CP,), jnp.int32),
            pltpu.VMEM((CP,), jnp.int32),
            pltpu.VMEM((CP, D), jnp.float32),
            pltpu.VMEM((ZROWS, D), jnp.float32),
            pltpu.VMEM_SHARED((NPAD, D), jnp.float32),
            pltpu.SemaphoreType.DMA,
        ],
    )
    sc_att = pl.kernel(
        _sc_att_body,
        out_type=(_f32(HEADS * E), _f32(NW * HEADS * N)),
        mesh=mesh,
        compiler_params=cp,
        scratch_types=[
            pltpu.VMEM((CP,), jnp.int32),
            pltpu.VMEM((CP,), jnp.int32),
            pltpu.VMEM((CP,), jnp.float32),
            pltpu.VMEM((N,), jnp.float32),
            pltpu.VMEM((NT,), jnp.float32),
            pltpu.VMEM((HEADS * 16,), jnp.float32),
            pltpu.VMEM((NT,), jnp.float32),
        ],
    )
    sc_gatmm = pl.kernel(
        _sc_gatmm_body,
        out_type=_f32(NC, HEADS, NPAD, D),
        mesh=mesh,
        compiler_params=cp,
        scratch_types=[
            pltpu.VMEM((CP,), jnp.int32),
            pltpu.VMEM((CP,), jnp.int32),
            pltpu.VMEM((CP,), jnp.float32),
            pltpu.VMEM((CP, D), jnp.float32),
            pltpu.VMEM((ZROWS, D), jnp.float32),
            pltpu.VMEM_SHARED((NPAD, D), jnp.float32),
            pltpu.SemaphoreType.DMA,
        ],
    )
    sc_sage = pl.kernel(
        _sc_sage_body,
        out_type=_f32(NC, HEADS, NPAD, D),
        mesh=mesh,
        compiler_params=cp,
        scratch_types=[
            pltpu.VMEM((CP,), jnp.int32),
            pltpu.VMEM((CP,), jnp.int32),
            pltpu.VMEM((CP, D), jnp.float32),
            pltpu.VMEM((ZROWS, D), jnp.float32),
            pltpu.VMEM_SHARED((NPAD, D), jnp.float32),
            pltpu.SemaphoreType.DMA,
        ],
    )
    sc_mlp = pl.kernel(
        _sc_mlp_body,
        out_type=_f32(E),
        mesh=mesh,
        compiler_params=cp,
        scratch_types=[
            pltpu.VMEM((CP,), jnp.int32),
            pltpu.VMEM((CP,), jnp.int32),
            pltpu.VMEM((CP, D), jnp.float32),
            pltpu.VMEM((CP, D), jnp.float32),
            pltpu.VMEM((CP * 16,), jnp.float32),
            pltpu.VMEM((CP,), jnp.float32),
            pltpu.VMEM((D,), jnp.float32),
            pltpu.VMEM((16,), jnp.float32),
            pltpu.SemaphoreType.DMA,
            pltpu.SemaphoreType.DMA,
        ],
    )
    return sc_indeg, sc_gcn, sc_att, sc_gatmm, sc_sage, sc_mlp


BM = 512
GRID = (N + BM - 1) // BM


def _dot(a, b):
    return jnp.dot(a, b, preferred_element_type=jnp.float32)


def _tc1_body(x_ref, w_ref, degp_ref, xw_ref, xws_ref):
    indeg = jnp.sum(degp_ref[...], axis=0)
    dinv = lax.rsqrt(indeg + 1.0)
    xw = _dot(x_ref[...], w_ref[...])
    xw_ref[...] = xw
    xws_ref[...] = xw * dinv[:, None]


def _tc1(x, w_gcn, degp):
    return pl.pallas_call(
        _tc1_body,
        grid=(GRID,),
        in_specs=[
            pl.BlockSpec((BM, D), lambda i: (i, 0)),
            pl.BlockSpec((D, HID), lambda i: (0, 0)),
            pl.BlockSpec((NW, BM), lambda i: (0, i)),
        ],
        out_specs=[
            pl.BlockSpec((BM, HID), lambda i: (i, 0)),
            pl.BlockSpec((BM, HID), lambda i: (i, 0)),
        ],
        out_shape=[_f32(N, HID), _f32(N, HID)],
    )(x, w_gcn, degp)


def _tc3_body(p_ref, xw_ref, degp_ref, bgcn_ref, wgat_ref, asrc_ref, adst_ref,
              xh0_ref, xh1_ref, xh2_ref, xh3_ref, asT_ref, adT_ref):
    indeg = jnp.sum(degp_ref[...], axis=0)
    dinv = lax.rsqrt(indeg + 1.0)
    ps = p_ref[0] + p_ref[1]
    h1 = jax.nn.relu(dinv[:, None] * ps + (dinv * dinv)[:, None] * xw_ref[...]
                     + bgcn_ref[...])
    xh = _dot(h1, wgat_ref[...])
    asrc = asrc_ref[...]
    adst = adst_ref[...]
    xh_refs = [xh0_ref, xh1_ref, xh2_ref, xh3_ref]
    a_s = []
    a_d = []
    for h in range(HEADS):
        xhh = xh[:, h * HID:(h + 1) * HID]
        xh_refs[h][...] = xhh
        a_s.append(jnp.sum(xhh * asrc[h][None, :], axis=1).reshape(1, BM))
        a_d.append(jnp.sum(xhh * adst[h][None, :], axis=1).reshape(1, BM))
    asT_ref[...] = jnp.concatenate(a_s, axis=0)
    adT_ref[...] = jnp.concatenate(a_d, axis=0)


def _tc3(p, xw, degp, b_gcn, w_gat, att_src, att_dst):
    return pl.pallas_call(
        _tc3_body,
        grid=(GRID,),
        in_specs=[
            pl.BlockSpec((NC, BM, HID), lambda i: (0, i, 0)),
            pl.BlockSpec((BM, HID), lambda i: (i, 0)),
            pl.BlockSpec((NW, BM), lambda i: (0, i)),
            pl.BlockSpec((1, HID), lambda i: (0, 0)),
            pl.BlockSpec((HID, HEADS * HID), lambda i: (0, 0)),
            pl.BlockSpec((HEADS, HID), lambda i: (0, 0)),
            pl.BlockSpec((HEADS, HID), lambda i: (0, 0)),
        ],
        out_specs=[pl.BlockSpec((BM, HID), lambda i: (i, 0))] * 4
        + [pl.BlockSpec((HEADS, BM), lambda i: (0, i))] * 2,
        out_shape=[_f32(N, HID)] * 4 + [_f32(HEADS, N)] * 2,
    )(p, xw, degp, b_gcn, w_gat, att_src, att_dst)


def _tc5_body(num_ref, denp_ref, asT_ref, adT_ref, m_ref,
              xh0_ref, xh1_ref, xh2_ref, xh3_ref, bgat_ref,
              h20_ref, h21_ref, h22_ref, h23_ref):
    denp = jnp.sum(denp_ref[...], axis=0)
    a = asT_ref[...] + adT_ref[...]
    e = jnp.where(a >= 0, a, 0.2 * a) - m_ref[...]
    exs = jnp.exp(e)
    xh_refs = [xh0_ref, xh1_ref, xh2_ref, xh3_ref]
    h2_refs = [h20_ref, h21_ref, h22_ref, h23_ref]
    bgat = bgat_ref[...]
    for h in range(HEADS):
        den = denp[h] + exs[h]
        nm = num_ref[0, h] + num_ref[1, h] + exs[h][:, None] * xh_refs[h][...]
        h2 = nm / (den[:, None] + 1e-16) + bgat[:, h * HID:(h + 1) * HID]
        h2_refs[h][...] = jax.nn.relu(h2)


def _tc5(num, denp, asT, adT, m, xhs, b_gat):
    return pl.pallas_call(
        _tc5_body,
        grid=(GRID,),
        in_specs=[
            pl.BlockSpec((NC, HEADS, BM, HID), lambda i: (0, 0, i, 0)),
            pl.BlockSpec((NW, HEADS, BM), lambda i: (0, 0, i)),
            pl.BlockSpec((HEADS, BM), lambda i: (0, i)),
            pl.BlockSpec((HEADS, BM), lambda i: (0, i)),
            pl.BlockSpec((HEADS, 1), lambda i: (0, 0)),
        ] + [pl.BlockSpec((BM, HID), lambda i: (i, 0))] * 4
        + [pl.BlockSpec((1, HEADS * HID), lambda i: (0, 0))],
        out_specs=[pl.BlockSpec((BM, HID), lambda i: (i, 0))] * 4,
        out_shape=[_f32(N, HID)] * 4,
    )(num, denp, asT, adT, m, *xhs, b_gat)


def _tc7_body(sp_ref, h20_ref, h21_ref, h22_ref, h23_ref, degp_ref,
              wsl_ref, wsr_ref, bsage_ref, w1_ref, b1_ref, u_ref, v_ref):
    indeg = jnp.sum(degp_ref[...], axis=0)
    invd = 1.0 / jnp.maximum(indeg, 1.0)
    h2_refs = [h20_ref, h21_ref, h22_ref, h23_ref]
    acc = jnp.broadcast_to(bsage_ref[...], (BM, HID))
    wsl = wsl_ref[...]
    wsr = wsr_ref[...]
    for h in range(HEADS):
        aggh = (sp_ref[0, h] + sp_ref[1, h]) * invd[:, None]
        acc = acc + _dot(aggh, wsl[h * HID:(h + 1) * HID])
        acc = acc + _dot(h2_refs[h][...], wsr[h * HID:(h + 1) * HID])
    h3 = jax.nn.relu(acc)
    w1 = w1_ref[...]
    u_ref[...] = _dot(h3, w1[:HID])
    v_ref[...] = _dot(h3, w1[HID:]) + b1_ref[...]


def _tc7(sp, h2s, degp, w_sage_l, w_sage_r, b_sage, w1, b1):
    return pl.pallas_call(
        _tc7_body,
        grid=(GRID,),
        in_specs=[
            pl.BlockSpec((NC, HEADS, BM, HID), lambda i: (0, 0, i, 0)),
        ] + [pl.BlockSpec((BM, HID), lambda i: (i, 0))] * 4 + [
            pl.BlockSpec((NW, BM), lambda i: (0, i)),
            pl.BlockSpec((HEADS * HID, HID), lambda i: (0, 0)),
            pl.BlockSpec((HEADS * HID, HID), lambda i: (0, 0)),
            pl.BlockSpec((1, HID), lambda i: (0, 0)),
            pl.BlockSpec((2 * HID, HID), lambda i: (0, 0)),
            pl.BlockSpec((1, HID), lambda i: (0, 0)),
        ],
        out_specs=[pl.BlockSpec((BM, HID), lambda i: (i, 0))] * 2,
        out_shape=[_f32(N, HID)] * 2,
    )(sp, *h2s, degp, w_sage_l, w_sage_r, b_sage, w1, b1)


def kernel(x, edge_index, W_gcn, b_gcn, W_gat, att_src, att_dst, b_gat,
           W_sage_l, W_sage_r, b_sage, W1, b1, W2, b2):
    src = edge_index[0]
    dst = edge_index[1]
    _sc_indeg, _sc_gcn, _sc_att, _sc_gatmm, _sc_sage, _sc_mlp = _sc_kernels()

    degp = _sc_indeg(dst).reshape(NW, N)
    xw, xws = _tc1(x, W_gcn, degp)
    p = _sc_gcn(src, dst, xws)
    xh0, xh1, xh2, xh3, asT, adT = _tc3(
        p[:, :N], xw, degp, b_gcn.reshape(1, -1), W_gat, att_src, att_dst)
    M = jnp.maximum(jnp.max(asT, axis=1) + jnp.max(adT, axis=1), 0.0)
    m64 = jnp.repeat(M, 16)
    exf, denf = _sc_att(src, dst, m64, asT.reshape(-1), adT.reshape(-1))
    num = _sc_gatmm(src, dst, exf, xh0, xh1, xh2, xh3)
    denp = denf.reshape(NW, HEADS, N)
    h2s = _tc5(num[:, :, :N], denp, asT, adT, M.reshape(HEADS, 1),
               [xh0, xh1, xh2, xh3], b_gat.reshape(1, -1))
    sp = _sc_sage(src, dst, *h2s)
    u, v = _tc7(sp[:, :, :N], h2s, degp, W_sage_l, W_sage_r,
                b_sage.reshape(1, -1), W1, b1.reshape(1, -1))
    pred = _sc_mlp(src, dst, u, v, W2.reshape(-1), jnp.pad(b2, (0, 15)))
    return pred


# speedup vs baseline: 8.6461x; 1.0317x over previous
import functools

import jax
import jax.numpy as jnp
from jax import lax
from jax.experimental import pallas as pl
from jax.experimental.pallas import tpu as pltpu
from jax.experimental.pallas import tpu_sc as plsc

N = 10000
E = 160000
D = 128
HID = 128
HEADS = 4

NC = 2
NS = 16
NW = NC * NS
NT = N + 16
NPAD = 10240
EPT = E // NW
CHK = 200
CP = CHK + 8
NCHUNKS = EPT // CHK
STRIPE = NPAD // NS
ZROWS = 40


def _f32(*shape):
    return jax.ShapeDtypeStruct(shape, jnp.float32)


def _wid():
    return lax.axis_index("c") * NS + lax.axis_index("s")


def _iota16():
    return lax.iota(jnp.int32, 16)


def _zero_vmem(ref, nwords):
    z = jnp.zeros((16,), jnp.float32)

    def body(i, c):
        ref[pl.ds(i * 16, 16)] = z
        return c

    lax.fori_loop(0, nwords // 16, body, 0)


def _zero_zbuf(zbuf):
    z = jnp.zeros((16,), jnp.float32)

    def body(i, c):
        r = i // 8
        k = i % 8
        zbuf[r, pl.ds(k * 16, 16)] = z
        return c

    lax.fori_loop(0, ZROWS * 8, body, 0)


def _zero_stripe(acc, base, zbuf):
    def body(i, c):
        pltpu.sync_copy(zbuf, acc.at[pl.ds(base + i * ZROWS, ZROWS)])
        return c

    lax.fori_loop(0, STRIPE // ZROWS, body, 0)


def _load_chunk_hbm(hbm, off, buf, pad_val):
    pltpu.sync_copy(hbm.at[pl.ds(off, CHK)], buf.at[pl.ds(0, CHK)])
    t = buf[pl.ds(CHK - 8, 16)]
    pv = jnp.full((16,), pad_val, jnp.int32)
    buf[pl.ds(CHK - 8, 16)] = jnp.where(_iota16() < 8, t, pv)


def _sc_indeg_body(dst_hbm, out_hbm, dst_res, hist):
    w = _wid()
    _zero_vmem(hist, NT)
    pltpu.sync_copy(dst_hbm.at[pl.ds(w * EPT, EPT)], dst_res.at[pl.ds(0, EPT)])
    ones = jnp.ones((16,), jnp.float32)
    trash = jnp.full((16,), N, jnp.int32)
    lanes = _iota16()

    @plsc.parallel_loop(0, (EPT + 15) // 16, unroll=4)
    def body(g):
        idx = dst_res[pl.ds(g * 16, 16)]
        m = (g * 16 + lanes) < EPT
        plsc.addupdate_scatter(hist, [jnp.where(m, idx, trash)], ones)
    pltpu.sync_copy(hist.at[pl.ds(0, N)], out_hbm.at[pl.ds(w * N, N)])


def _sc_gcn_body(src_hbm, dst_hbm, xws_hbm, out_hbm,
            sbuf, dbuf, rowbuf, zbuf, acc, sem):
    cid = lax.axis_index("c")
    sid = lax.axis_index("s")
    w = cid * NS + sid
    base = sid * STRIPE
    _zero_zbuf(zbuf)
    _zero_stripe(acc, base, zbuf)
    plsc.subcore_barrier()

    def chunk(c, carry):
        eb = w * EPT + c * CHK
        _load_chunk_hbm(src_hbm, eb, sbuf, 0)
        _load_chunk_hbm(dst_hbm, eb, dbuf, N)
        pltpu.async_copy(xws_hbm.at[sbuf], rowbuf, sem).wait()
        pltpu.sync_copy(rowbuf, acc.at[dbuf], add=True)
        return carry

    lax.fori_loop(0, NCHUNKS, chunk, 0)
    plsc.subcore_barrier()
    pltpu.sync_copy(acc.at[pl.ds(base, STRIPE)],
                    out_hbm.at[cid, pl.ds(base, STRIPE)])


def _sc_att_body(src_hbm, dst_hbm, m_hbm, asT_hbm, adT_hbm,
            ex_hbm, den_hbm,
            s_all, d_all, exbuf, as_buf, ad_buf, mbuf, den_loc):
    cid = lax.axis_index("c")
    sid = lax.axis_index("s")
    w = cid * NS + sid
    pltpu.sync_copy(m_hbm, mbuf)
    pltpu.sync_copy(src_hbm.at[pl.ds(w * EPT, EPT)], s_all.at[pl.ds(0, EPT)])
    pltpu.sync_copy(dst_hbm.at[pl.ds(w * EPT, EPT)], d_all.at[pl.ds(0, EPT)])
    lanes = _iota16()
    ts = s_all[pl.ds(EPT - 8, 16)]
    s_all[pl.ds(EPT - 8, 16)] = jnp.where(
        lanes < 8, ts, jnp.zeros((16,), jnp.int32))
    td = d_all[pl.ds(EPT - 8, 16)]
    d_all[pl.ds(EPT - 8, 16)] = jnp.where(
        lanes < 8, td, jnp.full((16,), N, jnp.int32))

    for h in range(HEADS):
        pltpu.sync_copy(asT_hbm.at[pl.ds(h * N, N)], as_buf)
        pltpu.sync_copy(adT_hbm.at[pl.ds(h * N, N)], ad_buf.at[pl.ds(0, N)])
        ad_buf[pl.ds(N, 16)] = jnp.zeros((16,), jnp.float32)
        _zero_vmem(den_loc, NT)
        mh = mbuf[pl.ds(h * 16, 16)]

        @plsc.parallel_loop(0, (EPT + 15) // 16, unroll=4)
        def exg(g):
            si = s_all[pl.ds(g * 16, 16)]
            di = d_all[pl.ds(g * 16, 16)]
            e = (plsc.load_gather(as_buf, [si])
                 + plsc.load_gather(ad_buf, [di]))
            e = jnp.where(e >= 0, e, 0.2 * e)
            ex = jnp.exp(e - mh)
            exbuf[pl.ds(g * 16, 16)] = ex
            plsc.addupdate_scatter(den_loc, [di], ex)

        pltpu.sync_copy(exbuf.at[pl.ds(0, EPT)],
                        ex_hbm.at[pl.ds(h * E + w * EPT, EPT)])
        pltpu.sync_copy(den_loc.at[pl.ds(0, N)],
                        den_hbm.at[pl.ds((w * HEADS + h) * N, N)])


def _sc_gatmm_body(src_hbm, dst_hbm, ex_hbm,
            xh0_hbm, xh1_hbm, xh2_hbm, xh3_hbm, num_hbm,
            sbuf, dbuf, exbuf, rowbuf, zbuf, acc, sem):
    cid = lax.axis_index("c")
    sid = lax.axis_index("s")
    w = cid * NS + sid
    base = sid * STRIPE
    _zero_zbuf(zbuf)
    xh_hbms = [xh0_hbm, xh1_hbm, xh2_hbm, xh3_hbm]
    zf = jnp.zeros((16,), jnp.float32)

    for h in range(HEADS):
        _zero_stripe(acc, base, zbuf)
        plsc.subcore_barrier()
        pltpu.sync_copy(ex_hbm.at[pl.ds(h * E + w * EPT, EPT)],
                        exbuf.at[pl.ds(0, EPT)])
        exbuf[pl.ds(EPT, 16)] = zf

        def chunk(c, carry):
            eb = w * EPT + c * CHK
            _load_chunk_hbm(src_hbm, eb, sbuf, 0)
            _load_chunk_hbm(dst_hbm, eb, dbuf, N)
            pltpu.async_copy(xh_hbms[h].at[sbuf], rowbuf, sem).wait()

            @plsc.parallel_loop(0, CP, unroll=4)
            def wrow(i):
                ww = plsc.load_gather(
                    exbuf, [jnp.zeros((16,), jnp.int32) + (c * CHK + i)])
                for r in range(D // 16):
                    rowbuf[i, pl.ds(r * 16, 16)] = (
                        rowbuf[i, pl.ds(r * 16, 16)] * ww)
            pltpu.sync_copy(rowbuf, acc.at[dbuf], add=True)
            return carry

        lax.fori_loop(0, NCHUNKS, chunk, 0)
        plsc.subcore_barrier()
        pltpu.sync_copy(acc.at[pl.ds(base, STRIPE)],
                        num_hbm.at[cid, h, pl.ds(base, STRIPE)])


def _sc_sage_body(src_hbm, dst_hbm, h20_hbm, h21_hbm, h22_hbm, h23_hbm, out_hbm,
             sbuf, dbuf, rowbuf, zbuf, acc, sem):
    cid = lax.axis_index("c")
    sid = lax.axis_index("s")
    w = cid * NS + sid
    base = sid * STRIPE
    _zero_zbuf(zbuf)
    h2_hbms = [h20_hbm, h21_hbm, h22_hbm, h23_hbm]

    for h in range(HEADS):
        _zero_stripe(acc, base, zbuf)
        plsc.subcore_barrier()

        def chunk(c, carry):
            eb = w * EPT + c * CHK
            _load_chunk_hbm(src_hbm, eb, sbuf, 0)
            _load_chunk_hbm(dst_hbm, eb, dbuf, N)
            pltpu.async_copy(h2_hbms[h].at[sbuf], rowbuf, sem).wait()
            pltpu.sync_copy(rowbuf, acc.at[dbuf], add=True)
            return carry

        lax.fori_loop(0, NCHUNKS, chunk, 0)
        plsc.subcore_barrier()
        pltpu.sync_copy(acc.at[pl.ds(base, STRIPE)],
                        out_hbm.at[cid, h, pl.ds(base, STRIPE)])


def _sc_mlp_body(src_hbm, dst_hbm, u_hbm, v_hbm, w2_hbm, b2_hbm, out_hbm,
            sbuf, dbuf, ubuf, vbuf, accbuf, predbuf,
            w2buf, b2buf, sem1, sem2):
    cid = lax.axis_index("c")
    sid = lax.axis_index("s")
    w = cid * NS + sid
    pltpu.sync_copy(w2_hbm, w2buf)
    pltpu.sync_copy(b2_hbm, b2buf)
    w2v = [w2buf[pl.ds(r * 16, 16)] for r in range(D // 16)]
    b2v = b2buf[...]
    lanes = _iota16()

    def chunk(c, carry):
        eb = w * EPT + c * CHK
        _load_chunk_hbm(src_hbm, eb, sbuf, 0)
        _load_chunk_hbm(dst_hbm, eb, dbuf, 0)
        cp1 = pltpu.async_copy(u_hbm.at[sbuf], ubuf, sem1)
        cp2 = pltpu.async_copy(v_hbm.at[dbuf], vbuf, sem2)
        cp1.wait()
        cp2.wait()

        @plsc.parallel_loop(0, CP, unroll=4)
        def edge(i):
            acc = jnp.zeros((16,), jnp.float32)
            for r in range(D // 16):
                z = jnp.maximum(
                    ubuf[i, pl.ds(r * 16, 16)] + vbuf[i, pl.ds(r * 16, 16)], 0.0)
                acc = acc + z * w2v[r]
            accbuf[pl.ds(i * 16, 16)] = acc

        @plsc.parallel_loop(0, CP // 16, unroll=2)
        def grp(g):
            tot = jnp.zeros((16,), jnp.float32)
            rowbase = (g * 16 + lanes) * 16
            for r in range(16):
                tot = tot + plsc.load_gather(accbuf, [rowbase + r])
            s = tot + b2v
            predbuf[pl.ds(g * 16, 16)] = 1.0 / (1.0 + jnp.exp(-s))
        pltpu.sync_copy(predbuf.at[pl.ds(0, CHK)],
                        out_hbm.at[pl.ds(w * EPT + c * CHK, CHK)])
        return carry

    lax.fori_loop(0, NCHUNKS, chunk, 0)


@functools.lru_cache(maxsize=None)
def _sc_kernels():
    mesh = plsc.VectorSubcoreMesh(core_axis_name="c", subcore_axis_name="s")
    cp = pltpu.CompilerParams(needs_layout_passes=False)
    sc_indeg = pl.kernel(
        _sc_indeg_body,
        out_type=_f32(NW * N),
        mesh=mesh,
        compiler_params=cp,
        scratch_types=[
            pltpu.VMEM((EPT + 16,), jnp.int32),
            pltpu.VMEM((NT,), jnp.float32),
        ],
    )
    sc_gcn = pl.kernel(
        _sc_gcn_body,
        out_type=_f32(NC, NPAD, D),
        mesh=mesh,
        compiler_params=cp,
        scratch_types=[
            pltpu.VMEM((CP,), jnp.int32),
            pltpu.VMEM((CP,), jnp.int32),
            pltpu.VMEM((CP, D), jnp.float32),
            pltpu.VMEM((ZROWS, D), jnp.float32),
            pltpu.VMEM_SHARED((NPAD, D), jnp.float32),
            pltpu.SemaphoreType.DMA,
        ],
    )
    sc_att = pl.kernel(
        _sc_att_body,
        out_type=(_f32(HEADS * E), _f32(NW * HEADS * N)),
        mesh=mesh,
        compiler_params=cp,
        scratch_types=[
            pltpu.VMEM((EPT + 16,), jnp.int32),
            pltpu.VMEM((EPT + 16,), jnp.int32),
            pltpu.VMEM((EPT + 16,), jnp.float32),
            pltpu.VMEM((N,), jnp.float32),
            pltpu.VMEM((NT,), jnp.float32),
            pltpu.VMEM((HEADS * 16,), jnp.float32),
            pltpu.VMEM((NT,), jnp.float32),
        ],
    )
    sc_gatmm = pl.kernel(
        _sc_gatmm_body,
        out_type=_f32(NC, HEADS, NPAD, D),
        mesh=mesh,
        compiler_params=cp,
        scratch_types=[
            pltpu.VMEM((CP,), jnp.int32),
            pltpu.VMEM((CP,), jnp.int32),
            pltpu.VMEM((EPT + 16,), jnp.float32),
            pltpu.VMEM((CP, D), jnp.float32),
            pltpu.VMEM((ZROWS, D), jnp.float32),
            pltpu.VMEM_SHARED((NPAD, D), jnp.float32),
            pltpu.SemaphoreType.DMA,
        ],
    )
    sc_sage = pl.kernel(
        _sc_sage_body,
        out_type=_f32(NC, HEADS, NPAD, D),
        mesh=mesh,
        compiler_params=cp,
        scratch_types=[
            pltpu.VMEM((CP,), jnp.int32),
            pltpu.VMEM((CP,), jnp.int32),
            pltpu.VMEM((CP, D), jnp.float32),
            pltpu.VMEM((ZROWS, D), jnp.float32),
            pltpu.VMEM_SHARED((NPAD, D), jnp.float32),
            pltpu.SemaphoreType.DMA,
        ],
    )
    sc_mlp = pl.kernel(
        _sc_mlp_body,
        out_type=_f32(E),
        mesh=mesh,
        compiler_params=cp,
        scratch_types=[
            pltpu.VMEM((CP,), jnp.int32),
            pltpu.VMEM((CP,), jnp.int32),
            pltpu.VMEM((CP, D), jnp.float32),
            pltpu.VMEM((CP, D), jnp.float32),
            pltpu.VMEM((CP * 16,), jnp.float32),
            pltpu.VMEM((CP,), jnp.float32),
            pltpu.VMEM((D,), jnp.float32),
            pltpu.VMEM((16,), jnp.float32),
            pltpu.SemaphoreType.DMA,
            pltpu.SemaphoreType.DMA,
        ],
    )
    return sc_indeg, sc_gcn, sc_att, sc_gatmm, sc_sage, sc_mlp


BM = 512
GRID = (N + BM - 1) // BM


def _dot(a, b):
    return jnp.dot(a, b, preferred_element_type=jnp.float32)


def _tc1_body(x_ref, w_ref, degp_ref, xw_ref, xws_ref):
    indeg = jnp.sum(degp_ref[...], axis=0)
    dinv = lax.rsqrt(indeg + 1.0)
    xw = _dot(x_ref[...], w_ref[...])
    xw_ref[...] = xw
    xws_ref[...] = xw * dinv[:, None]


def _tc1(x, w_gcn, degp):
    return pl.pallas_call(
        _tc1_body,
        grid=(GRID,),
        in_specs=[
            pl.BlockSpec((BM, D), lambda i: (i, 0)),
            pl.BlockSpec((D, HID), lambda i: (0, 0)),
            pl.BlockSpec((NW, BM), lambda i: (0, i)),
        ],
        out_specs=[
            pl.BlockSpec((BM, HID), lambda i: (i, 0)),
            pl.BlockSpec((BM, HID), lambda i: (i, 0)),
        ],
        out_shape=[_f32(N, HID), _f32(N, HID)],
    )(x, w_gcn, degp)


def _tc3_body(p_ref, xw_ref, degp_ref, bgcn_ref, wgat_ref, asrc_ref, adst_ref,
              xh0_ref, xh1_ref, xh2_ref, xh3_ref, asT_ref, adT_ref):
    indeg = jnp.sum(degp_ref[...], axis=0)
    dinv = lax.rsqrt(indeg + 1.0)
    ps = p_ref[0] + p_ref[1]
    h1 = jax.nn.relu(dinv[:, None] * ps + (dinv * dinv)[:, None] * xw_ref[...]
                     + bgcn_ref[...])
    xh = _dot(h1, wgat_ref[...])
    asrc = asrc_ref[...]
    adst = adst_ref[...]
    xh_refs = [xh0_ref, xh1_ref, xh2_ref, xh3_ref]
    a_s = []
    a_d = []
    for h in range(HEADS):
        xhh = xh[:, h * HID:(h + 1) * HID]
        xh_refs[h][...] = xhh
        a_s.append(jnp.sum(xhh * asrc[h][None, :], axis=1).reshape(1, BM))
        a_d.append(jnp.sum(xhh * adst[h][None, :], axis=1).reshape(1, BM))
    asT_ref[...] = jnp.concatenate(a_s, axis=0)
    adT_ref[...] = jnp.concatenate(a_d, axis=0)


def _tc3(p, xw, degp, b_gcn, w_gat, att_src, att_dst):
    return pl.pallas_call(
        _tc3_body,
        grid=(GRID,),
        in_specs=[
            pl.BlockSpec((NC, BM, HID), lambda i: (0, i, 0)),
            pl.BlockSpec((BM, HID), lambda i: (i, 0)),
            pl.BlockSpec((NW, BM), lambda i: (0, i)),
            pl.BlockSpec((1, HID), lambda i: (0, 0)),
            pl.BlockSpec((HID, HEADS * HID), lambda i: (0, 0)),
            pl.BlockSpec((HEADS, HID), lambda i: (0, 0)),
            pl.BlockSpec((HEADS, HID), lambda i: (0, 0)),
        ],
        out_specs=[pl.BlockSpec((BM, HID), lambda i: (i, 0))] * 4
        + [pl.BlockSpec((HEADS, BM), lambda i: (0, i))] * 2,
        out_shape=[_f32(N, HID)] * 4 + [_f32(HEADS, N)] * 2,
    )(p, xw, degp, b_gcn, w_gat, att_src, att_dst)


def _tc5_body(num_ref, denp_ref, asT_ref, adT_ref, m_ref,
              xh0_ref, xh1_ref, xh2_ref, xh3_ref, bgat_ref,
              h20_ref, h21_ref, h22_ref, h23_ref):
    denp = jnp.sum(denp_ref[...], axis=0)
    a = asT_ref[...] + adT_ref[...]
    e = jnp.where(a >= 0, a, 0.2 * a) - m_ref[...]
    exs = jnp.exp(e)
    xh_refs = [xh0_ref, xh1_ref, xh2_ref, xh3_ref]
    h2_refs = [h20_ref, h21_ref, h22_ref, h23_ref]
    bgat = bgat_ref[...]
    for h in range(HEADS):
        den = denp[h] + exs[h]
        nm = num_ref[0, h] + num_ref[1, h] + exs[h][:, None] * xh_refs[h][...]
        h2 = nm / (den[:, None] + 1e-16) + bgat[:, h * HID:(h + 1) * HID]
        h2_refs[h][...] = jax.nn.relu(h2)


def _tc5(num, denp, asT, adT, m, xhs, b_gat):
    return pl.pallas_call(
        _tc5_body,
        grid=(GRID,),
        in_specs=[
            pl.BlockSpec((NC, HEADS, BM, HID), lambda i: (0, 0, i, 0)),
            pl.BlockSpec((NW, HEADS, BM), lambda i: (0, 0, i)),
            pl.BlockSpec((HEADS, BM), lambda i: (0, i)),
            pl.BlockSpec((HEADS, BM), lambda i: (0, i)),
            pl.BlockSpec((HEADS, 1), lambda i: (0, 0)),
        ] + [pl.BlockSpec((BM, HID), lambda i: (i, 0))] * 4
        + [pl.BlockSpec((1, HEADS * HID), lambda i: (0, 0))],
        out_specs=[pl.BlockSpec((BM, HID), lambda i: (i, 0))] * 4,
        out_shape=[_f32(N, HID)] * 4,
    )(num, denp, asT, adT, m, *xhs, b_gat)


def _tc7_body(sp_ref, h20_ref, h21_ref, h22_ref, h23_ref, degp_ref,
              wsl_ref, wsr_ref, bsage_ref, w1_ref, b1_ref, u_ref, v_ref):
    indeg = jnp.sum(degp_ref[...], axis=0)
    invd = 1.0 / jnp.maximum(indeg, 1.0)
    h2_refs = [h20_ref, h21_ref, h22_ref, h23_ref]
    acc = jnp.broadcast_to(bsage_ref[...], (BM, HID))
    wsl = wsl_ref[...]
    wsr = wsr_ref[...]
    for h in range(HEADS):
        aggh = (sp_ref[0, h] + sp_ref[1, h]) * invd[:, None]
        acc = acc + _dot(aggh, wsl[h * HID:(h + 1) * HID])
        acc = acc + _dot(h2_refs[h][...], wsr[h * HID:(h + 1) * HID])
    h3 = jax.nn.relu(acc)
    w1 = w1_ref[...]
    u_ref[...] = _dot(h3, w1[:HID])
    v_ref[...] = _dot(h3, w1[HID:]) + b1_ref[...]


def _tc7(sp, h2s, degp, w_sage_l, w_sage_r, b_sage, w1, b1):
    return pl.pallas_call(
        _tc7_body,
        grid=(GRID,),
        in_specs=[
            pl.BlockSpec((NC, HEADS, BM, HID), lambda i: (0, 0, i, 0)),
        ] + [pl.BlockSpec((BM, HID), lambda i: (i, 0))] * 4 + [
            pl.BlockSpec((NW, BM), lambda i: (0, i)),
            pl.BlockSpec((HEADS * HID, HID), lambda i: (0, 0)),
            pl.BlockSpec((HEADS * HID, HID), lambda i: (0, 0)),
            pl.BlockSpec((1, HID), lambda i: (0, 0)),
            pl.BlockSpec((2 * HID, HID), lambda i: (0, 0)),
            pl.BlockSpec((1, HID), lambda i: (0, 0)),
        ],
        out_specs=[pl.BlockSpec((BM, HID), lambda i: (i, 0))] * 2,
        out_shape=[_f32(N, HID)] * 2,
    )(sp, *h2s, degp, w_sage_l, w_sage_r, b_sage, w1, b1)


def kernel(x, edge_index, W_gcn, b_gcn, W_gat, att_src, att_dst, b_gat,
           W_sage_l, W_sage_r, b_sage, W1, b1, W2, b2):
    src = edge_index[0]
    dst = edge_index[1]
    _sc_indeg, _sc_gcn, _sc_att, _sc_gatmm, _sc_sage, _sc_mlp = _sc_kernels()

    degp = _sc_indeg(dst).reshape(NW, N)
    xw, xws = _tc1(x, W_gcn, degp)
    p = _sc_gcn(src, dst, xws)
    xh0, xh1, xh2, xh3, asT, adT = _tc3(
        p[:, :N], xw, degp, b_gcn.reshape(1, -1), W_gat, att_src, att_dst)
    M = jnp.maximum(jnp.max(asT, axis=1) + jnp.max(adT, axis=1), 0.0)
    m64 = jnp.repeat(M, 16)
    exf, denf = _sc_att(src, dst, m64, asT.reshape(-1), adT.reshape(-1))
    num = _sc_gatmm(src, dst, exf, xh0, xh1, xh2, xh3)
    denp = denf.reshape(NW, HEADS, N)
    h2s = _tc5(num[:, :, :N], denp, asT, adT, M.reshape(HEADS, 1),
               [xh0, xh1, xh2, xh3], b_gat.reshape(1, -1))
    sp = _sc_sage(src, dst, *h2s)
    u, v = _tc7(sp[:, :, :N], h2s, degp, W_sage_l, W_sage_r,
                b_sage.reshape(1, -1), W1, b1.reshape(1, -1))
    pred = _sc_mlp(src, dst, u, v, W2.reshape(-1), jnp.pad(b2, (0, 15)))
    return pred
